# baseline (speedup 1.0000x reference)
import functools

import jax
import jax.numpy as jnp
from jax import lax
from jax.experimental import pallas as pl
from jax.experimental.pallas import tpu as pltpu
from jax.experimental.pallas import tpu_sc as plsc

N_CORES = 2
N_SUBCORES = 16
N_TILES = N_CORES * N_SUBCORES
K = 80
M_BLK = 2000


def _sc_mesh():
    return plsc.VectorSubcoreMesh(core_axis_name="c", subcore_axis_name="s")


def _row_partition(N):
    rps = (N // 8 // N_SUBCORES) * 8
    tail = N - N_SUBCORES * rps
    return rps, tail


def _striped_rows_copy(src, dst, s, N):
    rps, tail = _row_partition(N)
    pltpu.sync_copy(src.at[pl.ds(s * rps, rps)], dst.at[pl.ds(s * rps, rps)])
    if tail:
        @pl.when(s == N_SUBCORES - 1)
        def _():
            pltpu.sync_copy(src.at[pl.ds(N_SUBCORES * rps, tail)],
                            dst.at[pl.ds(N_SUBCORES * rps, tail)])


NBUF = 5


def _make_deg_kernel(E, N):
    ept = E // N_TILES
    nch = ept // K

    @functools.partial(
        pl.kernel,
        out_type=jax.ShapeDtypeStruct((N_CORES, N, 16), jnp.float32),
        mesh=_sc_mesh(),
        compiler_params=pltpu.CompilerParams(use_tc_tiling_on_sc=False),
        scratch_types=[
            pltpu.VMEM((E // K // N_TILES, K), jnp.int32),
            pltpu.VMEM((K, 16), jnp.float32),
            pltpu.VMEM_SHARED((N, 16), jnp.float32),
            pltpu.SemaphoreType.DMA,
        ],
    )
    def deg_kernel(ei2_hbm, zero_hbm, ones_hbm, out_hbm,
                   didx, ones_v, acc_sh, ssem):
        c = lax.axis_index("c")
        s = lax.axis_index("s")
        wid = c * N_SUBCORES + s
        pltpu.sync_copy(ones_hbm, ones_v)
        pltpu.sync_copy(ei2_hbm.at[1].at[pl.ds(wid * nch, nch)], didx)
        _striped_rows_copy(zero_hbm, acc_sh, s, N)
        plsc.subcore_barrier()

        @pl.loop(0, nch, step=NBUF)
        def _(c0):
            for b in range(NBUF):
                pltpu.async_copy(ones_v, acc_sh.at[didx.at[c0 + b]], ssem,
                                 add=True)
            for b in range(NBUF):
                pltpu.make_async_copy(ones_v, acc_sh.at[didx.at[0]],
                                      ssem).wait()

        plsc.subcore_barrier()
        _striped_rows_copy(acc_sh, out_hbm.at[c], s, N)

    return deg_kernel


def _make_agg_kernel(E, N, D, nbuf, stream_idx=False):
    ept = E // N_TILES
    nch = ept // K
    main = nch - nch % nbuf
    ngrp = main // nbuf
    assert ept % K == 0 and main > 0
    nidx = 2 if stream_idx else 1
    idx_rows = nbuf if stream_idx else nch

    @functools.partial(
        pl.kernel,
        out_type=jax.ShapeDtypeStruct((N_CORES, N, D), jnp.float32),
        mesh=_sc_mesh(),
        compiler_params=pltpu.CompilerParams(use_tc_tiling_on_sc=False),
        scratch_types=[
            pltpu.VMEM((nidx, idx_rows, K), jnp.int32),
            pltpu.VMEM((nidx, idx_rows, K), jnp.int32),
            pltpu.VMEM((nbuf, K, D), jnp.float32),
            pltpu.VMEM_SHARED((N, D), jnp.float32),
            pltpu.SemaphoreType.DMA((nbuf,)),
            pltpu.SemaphoreType.DMA((nbuf,)),
            pltpu.SemaphoreType.DMA((2,)),
            pltpu.SemaphoreType.DMA((2,)),
        ],
    )
    def agg_kernel(g_hbm, ei2_hbm, zero_hbm, out_hbm,
                   sidx, didx, rows, acc_sh, gsem, ssem, is1, is2):
        c = lax.axis_index("c")
        s = lax.axis_index("s")
        wid = c * N_SUBCORES + s
        ebase = wid * nch
        ei0 = ei2_hbm.at[0]
        ei1 = ei2_hbm.at[1]
        if stream_idx:
            pltpu.sync_copy(ei0.at[pl.ds(ebase, nbuf)], sidx.at[0])
            pltpu.sync_copy(ei1.at[pl.ds(ebase, nbuf)], didx.at[0])
        else:
            pltpu.sync_copy(ei0.at[pl.ds(ebase, nch)], sidx.at[0])
            pltpu.sync_copy(ei1.at[pl.ds(ebase, nch)], didx.at[0])
        _striped_rows_copy(zero_hbm, acc_sh, s, N)
        plsc.subcore_barrier()

        @pl.loop(0, ngrp)
        def _(gp):
            c0 = gp * nbuf
            if stream_idx:
                slot = lax.rem(gp, 2)
                srow = sidx.at[slot]
                drow = didx.at[slot]
                @pl.when(gp > 0)
                def _():
                    pltpu.make_async_copy(ei0.at[pl.ds(ebase, nbuf)],
                                          sidx.at[slot], is1.at[slot]).wait()
                    pltpu.make_async_copy(ei1.at[pl.ds(ebase, nbuf)],
                                          didx.at[slot], is2.at[slot]).wait()
            else:
                srow = sidx.at[0].at[pl.ds(c0, nbuf)]
                drow = didx.at[0].at[pl.ds(c0, nbuf)]
            for b in range(nbuf):
                @pl.when(gp > 0)
                def _():
                    pltpu.make_async_copy(rows.at[b],
                                          acc_sh.at[didx.at[0].at[0]],
                                          ssem.at[b]).wait()
                pltpu.async_copy(g_hbm.at[srow.at[b]], rows.at[b],
                                 gsem.at[b])
            if stream_idx:
                @pl.when(gp + 1 < ngrp)
                def _():
                    nxt = 1 - slot
                    off = ebase + c0 + nbuf
                    pltpu.async_copy(ei0.at[pl.ds(off, nbuf)], sidx.at[nxt],
                                     is1.at[nxt])
                    pltpu.async_copy(ei1.at[pl.ds(off, nbuf)], didx.at[nxt],
                                     is2.at[nxt])
            for b in range(nbuf):
                pltpu.make_async_copy(g_hbm.at[srow.at[0]], rows.at[b],
                                      gsem.at[b]).wait()
                pltpu.async_copy(rows.at[b], acc_sh.at[drow.at[b]],
                                 ssem.at[b], add=True)

        for b in range(nbuf):
            pltpu.make_async_copy(rows.at[b], acc_sh.at[didx.at[0].at[0]],
                                  ssem.at[b]).wait()
        for t, ch in enumerate(range(main, nch)):
            if stream_idx:
                pltpu.sync_copy(ei0.at[ch + ebase], sidx.at[0].at[t])
                pltpu.sync_copy(ei1.at[ch + ebase], didx.at[0].at[t])
                srow_t = sidx.at[0].at[t]
                drow_t = didx.at[0].at[t]
            else:
                srow_t = sidx.at[0].at[ch]
                drow_t = didx.at[0].at[ch]
            pltpu.async_copy(g_hbm.at[srow_t], rows.at[t],
                             gsem.at[t]).wait()
            pltpu.sync_copy(rows.at[t], acc_sh.at[drow_t], add=True)
        plsc.subcore_barrier()
        _striped_rows_copy(acc_sh, out_hbm.at[c], s, N)

    return agg_kernel


def _dot(a, b):
    return lax.dot_general(a, b, (((1,), (0,)), ((), ())),
                           precision=lax.Precision.HIGHEST,
                           preferred_element_type=jnp.float32)


def _mm_scale_body(x_ref, w_ref, deg_ref, g_ref, dinv_ref):
    degsum = deg_ref[0] + deg_ref[1]
    dinv = lax.rsqrt(degsum[:, 0:1] + 1.0)
    dinv_ref[...] = dinv
    g_ref[...] = _dot(x_ref[...], w_ref[...]) * dinv


def _epi1_body(agg_ref, g1_ref, dinv_ref, b1_ref, w2_ref, g2_ref):
    dinv = dinv_ref[...]
    pre = dinv * (agg_ref[0] + agg_ref[1] + g1_ref[...]) + b1_ref[...]
    r = jnp.maximum(pre, 0.0)
    g2_ref[...] = _dot(r, w2_ref[...]) * dinv


def _epi2_body(agg_ref, g2_ref, dinv_ref, b2_ref, o_ref):
    o_ref[...] = (dinv_ref[...] * (agg_ref[0] + agg_ref[1] + g2_ref[...])
                  + b2_ref[...])


def kernel(x, edge_index, W1, b1, W2, b2):
    N, FEAT = x.shape
    HID = W1.shape[1]
    OUT = W2.shape[1]
    E = edge_index.shape[1]
    grid = (N // M_BLK,)

    ei2 = edge_index.reshape(2, E // K, K)
    z16 = jnp.zeros((N, 16), jnp.float32)
    ones = jnp.ones((K, 16), jnp.float32)
    zH = jnp.zeros((N, HID), jnp.float32)
    zO = jnp.zeros((N, OUT), jnp.float32)

    deg = _make_deg_kernel(E, N)(ei2, z16, ones)

    g1, dinv = pl.pallas_call(
        _mm_scale_body,
        grid=grid,
        in_specs=[pl.BlockSpec((M_BLK, FEAT), lambda i: (i, 0)),
                  pl.BlockSpec((FEAT, HID), lambda i: (0, 0)),
                  pl.BlockSpec((N_CORES, M_BLK, 16), lambda i: (0, i, 0))],
        out_specs=[pl.BlockSpec((M_BLK, HID), lambda i: (i, 0)),
                   pl.BlockSpec((M_BLK, 1), lambda i: (i, 0))],
        out_shape=[jax.ShapeDtypeStruct((N, HID), jnp.float32),
                   jax.ShapeDtypeStruct((N, 1), jnp.float32)],
    )(x, W1, deg)

    agg1 = _make_agg_kernel(E, N, HID, 4, stream_idx=True)(g1, ei2, zH)

    g2 = pl.pallas_call(
        _epi1_body,
        grid=grid,
        in_specs=[pl.BlockSpec((N_CORES, M_BLK, HID), lambda i: (0, i, 0)),
                  pl.BlockSpec((M_BLK, HID), lambda i: (i, 0)),
                  pl.BlockSpec((M_BLK, 1), lambda i: (i, 0)),
                  pl.BlockSpec((1, HID), lambda i: (0, 0)),
                  pl.BlockSpec((HID, OUT), lambda i: (0, 0))],
        out_specs=pl.BlockSpec((M_BLK, OUT), lambda i: (i, 0)),
        out_shape=jax.ShapeDtypeStruct((N, OUT), jnp.float32),
    )(agg1, g1, dinv, b1.reshape(1, HID), W2)

    agg2 = _make_agg_kernel(E, N, OUT, 10)(g2, ei2, zO)

    out = pl.pallas_call(
        _epi2_body,
        grid=grid,
        in_specs=[pl.BlockSpec((N_CORES, M_BLK, OUT), lambda i: (0, i, 0)),
                  pl.BlockSpec((M_BLK, OUT), lambda i: (i, 0)),
                  pl.BlockSpec((M_BLK, 1), lambda i: (i, 0)),
                  pl.BlockSpec((1, OUT), lambda i: (0, 0))],
        out_specs=pl.BlockSpec((M_BLK, OUT), lambda i: (i, 0)),
        out_shape=jax.ShapeDtypeStruct((N, OUT), jnp.float32),
    )(agg2, g2, dinv, b2.reshape(1, OUT))

    return out


# speedup vs baseline: 1.0039x; 1.0039x over previous
import functools

import jax
import jax.numpy as jnp
from jax import lax
from jax.experimental import pallas as pl
from jax.experimental.pallas import tpu as pltpu
from jax.experimental.pallas import tpu_sc as plsc

N_CORES = 2
N_SUBCORES = 16
N_TILES = N_CORES * N_SUBCORES
K = 80
M_BLK = 2000


def _sc_mesh():
    return plsc.VectorSubcoreMesh(core_axis_name="c", subcore_axis_name="s")


def _row_partition(N):
    rps = (N // 8 // N_SUBCORES) * 8
    tail = N - N_SUBCORES * rps
    return rps, tail


def _striped_rows_copy(src, dst, s, N):
    rps, tail = _row_partition(N)
    pltpu.sync_copy(src.at[pl.ds(s * rps, rps)], dst.at[pl.ds(s * rps, rps)])
    if tail:
        @pl.when(s == N_SUBCORES - 1)
        def _():
            pltpu.sync_copy(src.at[pl.ds(N_SUBCORES * rps, tail)],
                            dst.at[pl.ds(N_SUBCORES * rps, tail)])


NBUF = 5


def _make_deg_kernel(E, N):
    ept = E // N_TILES
    nch = ept // K

    @functools.partial(
        pl.kernel,
        out_type=jax.ShapeDtypeStruct((N_CORES, N, 16), jnp.float32),
        mesh=_sc_mesh(),
        compiler_params=pltpu.CompilerParams(use_tc_tiling_on_sc=False),
        scratch_types=[
            pltpu.VMEM((E // K // N_TILES, K), jnp.int32),
            pltpu.VMEM((K, 16), jnp.float32),
            pltpu.VMEM_SHARED((N, 16), jnp.float32),
            pltpu.SemaphoreType.DMA,
        ],
    )
    def deg_kernel(ei2_hbm, zero_hbm, ones_hbm, out_hbm,
                   didx, ones_v, acc_sh, ssem):
        c = lax.axis_index("c")
        s = lax.axis_index("s")
        wid = c * N_SUBCORES + s
        pltpu.sync_copy(ones_hbm, ones_v)
        pltpu.sync_copy(ei2_hbm.at[1].at[pl.ds(wid * nch, nch)], didx)
        _striped_rows_copy(zero_hbm, acc_sh, s, N)
        plsc.subcore_barrier()

        @pl.loop(0, nch, step=NBUF)
        def _(c0):
            for b in range(NBUF):
                pltpu.async_copy(ones_v, acc_sh.at[didx.at[c0 + b]], ssem,
                                 add=True)
            for b in range(NBUF):
                pltpu.make_async_copy(ones_v, acc_sh.at[didx.at[0]],
                                      ssem).wait()

        plsc.subcore_barrier()
        _striped_rows_copy(acc_sh, out_hbm.at[c], s, N)

    return deg_kernel


def _make_agg_kernel(E, N, D, nbuf, stream_idx=False):
    ept = E // N_TILES
    nch = ept // K
    main = nch - nch % nbuf
    ngrp = main // nbuf
    assert ept % K == 0 and main > 0
    nidx = 2 if stream_idx else 1
    idx_rows = nbuf if stream_idx else nch

    @functools.partial(
        pl.kernel,
        out_type=jax.ShapeDtypeStruct((N_CORES, N, D), jnp.float32),
        mesh=_sc_mesh(),
        compiler_params=pltpu.CompilerParams(use_tc_tiling_on_sc=False),
        scratch_types=[
            pltpu.VMEM((nidx, idx_rows, K), jnp.int32),
            pltpu.VMEM((nidx, idx_rows, K), jnp.int32),
            pltpu.VMEM((nbuf, K, D), jnp.float32),
            pltpu.VMEM_SHARED((N, D), jnp.float32),
            pltpu.SemaphoreType.DMA((nbuf,)),
            pltpu.SemaphoreType.DMA((nbuf,)),
            pltpu.SemaphoreType.DMA((2,)),
            pltpu.SemaphoreType.DMA((2,)),
        ],
    )
    def agg_kernel(g_hbm, ei2_hbm, zero_hbm, out_hbm,
                   sidx, didx, rows, acc_sh, gsem, ssem, is1, is2):
        c = lax.axis_index("c")
        s = lax.axis_index("s")
        wid = c * N_SUBCORES + s
        ebase = wid * nch
        ei0 = ei2_hbm.at[0]
        ei1 = ei2_hbm.at[1]
        if stream_idx:
            pltpu.sync_copy(ei0.at[pl.ds(ebase, nbuf)], sidx.at[0])
            pltpu.sync_copy(ei1.at[pl.ds(ebase, nbuf)], didx.at[0])
        else:
            pltpu.sync_copy(ei0.at[pl.ds(ebase, nch)], sidx.at[0])
            pltpu.sync_copy(ei1.at[pl.ds(ebase, nch)], didx.at[0])
        _striped_rows_copy(zero_hbm, acc_sh, s, N)
        plsc.subcore_barrier()

        @pl.loop(0, ngrp)
        def _(gp):
            c0 = gp * nbuf
            if stream_idx:
                slot = lax.rem(gp, 2)
                srow = sidx.at[slot]
                drow = didx.at[slot]
                @pl.when(gp > 0)
                def _():
                    pltpu.make_async_copy(ei0.at[pl.ds(ebase, nbuf)],
                                          sidx.at[slot], is1.at[slot]).wait()
                    pltpu.make_async_copy(ei1.at[pl.ds(ebase, nbuf)],
                                          didx.at[slot], is2.at[slot]).wait()
            else:
                srow = sidx.at[0].at[pl.ds(c0, nbuf)]
                drow = didx.at[0].at[pl.ds(c0, nbuf)]
            for b in range(nbuf):
                @pl.when(gp > 0)
                def _():
                    pltpu.make_async_copy(rows.at[b],
                                          acc_sh.at[didx.at[0].at[0]],
                                          ssem.at[b]).wait()
                pltpu.async_copy(g_hbm.at[srow.at[b]], rows.at[b],
                                 gsem.at[b])
            if stream_idx:
                @pl.when(gp + 1 < ngrp)
                def _():
                    nxt = 1 - slot
                    off = ebase + c0 + nbuf
                    pltpu.async_copy(ei0.at[pl.ds(off, nbuf)], sidx.at[nxt],
                                     is1.at[nxt])
                    pltpu.async_copy(ei1.at[pl.ds(off, nbuf)], didx.at[nxt],
                                     is2.at[nxt])
            for b in range(nbuf):
                pltpu.make_async_copy(g_hbm.at[srow.at[0]], rows.at[b],
                                      gsem.at[b]).wait()
                pltpu.async_copy(rows.at[b], acc_sh.at[drow.at[b]],
                                 ssem.at[b], add=True)

        for b in range(nbuf):
            pltpu.make_async_copy(rows.at[b], acc_sh.at[didx.at[0].at[0]],
                                  ssem.at[b]).wait()
        for t, ch in enumerate(range(main, nch)):
            if stream_idx:
                pltpu.sync_copy(ei0.at[ch + ebase], sidx.at[0].at[t])
                pltpu.sync_copy(ei1.at[ch + ebase], didx.at[0].at[t])
                srow_t = sidx.at[0].at[t]
                drow_t = didx.at[0].at[t]
            else:
                srow_t = sidx.at[0].at[ch]
                drow_t = didx.at[0].at[ch]
            pltpu.async_copy(g_hbm.at[srow_t], rows.at[t],
                             gsem.at[t]).wait()
            pltpu.sync_copy(rows.at[t], acc_sh.at[drow_t], add=True)
        plsc.subcore_barrier()
        _striped_rows_copy(acc_sh, out_hbm.at[c], s, N)

    return agg_kernel


def _dot(a, b):
    return lax.dot_general(a, b, (((1,), (0,)), ((), ())),
                           precision=lax.Precision.HIGHEST,
                           preferred_element_type=jnp.float32)


def _mm_scale_body(x_ref, w_ref, deg_ref, g_ref, dinv_ref):
    degsum = deg_ref[0] + deg_ref[1]
    dinv = lax.rsqrt(degsum[:, 0:1] + 1.0)
    dinv_ref[...] = dinv
    g_ref[...] = _dot(x_ref[...], w_ref[...]) * dinv


def _epi1_body(agg_ref, g1_ref, dinv_ref, b1_ref, w2_ref, g2_ref):
    dinv = dinv_ref[...]
    pre = dinv * (agg_ref[0] + agg_ref[1] + g1_ref[...]) + b1_ref[...]
    r = jnp.maximum(pre, 0.0)
    g2_ref[...] = _dot(r, w2_ref[...]) * dinv


def _epi2_body(agg_ref, g2_ref, dinv_ref, b2_ref, o_ref):
    o_ref[...] = (dinv_ref[...] * (agg_ref[0] + agg_ref[1] + g2_ref[...])
                  + b2_ref[...])


def kernel(x, edge_index, W1, b1, W2, b2):
    N, FEAT = x.shape
    HID = W1.shape[1]
    OUT = W2.shape[1]
    E = edge_index.shape[1]
    grid = (N // M_BLK,)

    ei2 = edge_index.reshape(2, E // K, K)
    z16 = jnp.zeros((N, 16), jnp.float32)
    ones = jnp.ones((K, 16), jnp.float32)
    zH = jnp.zeros((N, HID), jnp.float32)
    zO = jnp.zeros((N, OUT), jnp.float32)

    deg = _make_deg_kernel(E, N)(ei2, z16, ones)

    g1, dinv = pl.pallas_call(
        _mm_scale_body,
        grid=grid,
        in_specs=[pl.BlockSpec((M_BLK, FEAT), lambda i: (i, 0)),
                  pl.BlockSpec((FEAT, HID), lambda i: (0, 0)),
                  pl.BlockSpec((N_CORES, M_BLK, 16), lambda i: (0, i, 0))],
        out_specs=[pl.BlockSpec((M_BLK, HID), lambda i: (i, 0)),
                   pl.BlockSpec((M_BLK, 1), lambda i: (i, 0))],
        out_shape=[jax.ShapeDtypeStruct((N, HID), jnp.float32),
                   jax.ShapeDtypeStruct((N, 1), jnp.float32)],
    )(x, W1, deg)

    agg1 = _make_agg_kernel(E, N, HID, 4, stream_idx=True)(g1, ei2, zH)

    g2 = pl.pallas_call(
        _epi1_body,
        grid=grid,
        in_specs=[pl.BlockSpec((N_CORES, M_BLK, HID), lambda i: (0, i, 0)),
                  pl.BlockSpec((M_BLK, HID), lambda i: (i, 0)),
                  pl.BlockSpec((M_BLK, 1), lambda i: (i, 0)),
                  pl.BlockSpec((1, HID), lambda i: (0, 0)),
                  pl.BlockSpec((HID, OUT), lambda i: (0, 0))],
        out_specs=pl.BlockSpec((M_BLK, OUT), lambda i: (i, 0)),
        out_shape=jax.ShapeDtypeStruct((N, OUT), jnp.float32),
    )(agg1, g1, dinv, b1.reshape(1, HID), W2)

    agg2 = _make_agg_kernel(E, N, OUT, NBUF)(g2, ei2, zO)

    out = pl.pallas_call(
        _epi2_body,
        grid=grid,
        in_specs=[pl.BlockSpec((N_CORES, M_BLK, OUT), lambda i: (0, i, 0)),
                  pl.BlockSpec((M_BLK, OUT), lambda i: (i, 0)),
                  pl.BlockSpec((M_BLK, 1), lambda i: (i, 0)),
                  pl.BlockSpec((1, OUT), lambda i: (0, 0))],
        out_specs=pl.BlockSpec((M_BLK, OUT), lambda i: (i, 0)),
        out_shape=jax.ShapeDtypeStruct((N, OUT), jnp.float32),
    )(agg2, g2, dinv, b2.reshape(1, OUT))

    return out
